# baseline (speedup 1.0000x reference)
import functools

import jax
import jax.numpy as jnp
from jax import lax
from jax.experimental import pallas as pl
from jax.experimental.pallas import tpu as pltpu
from jax.experimental.pallas import tpu_sc as plsc

N = 10000
E = 160000
IN = 256
HID = 1024
OUT = 128
G = 16

NPAD = 10240
CW = 128
EB = 100
SB = 20
NSB = 5
NB_DEG = 50
NSUB = 16
NCORE = 2
ZROWS = NPAD // NSUB
NC1 = IN // CW
NC2 = HID // CW


@functools.cache
def _sc_mesh():
    return plsc.VectorSubcoreMesh(core_axis_name="c", subcore_axis_name="s",
                                  num_cores=NCORE, num_subcores=NSUB)


@functools.cache
def _make_deg():
    def body(dst_hbm, ones_hbm, zeros_hbm, out0, out1, dst_v, ones_v, acc, sem):
        c = lax.axis_index("c")
        s = lax.axis_index("s")
        pltpu.sync_copy(dst_hbm.at[c * NSUB + s], dst_v)
        pltpu.sync_copy(ones_hbm, ones_v)
        pltpu.sync_copy(zeros_hbm, acc.at[pl.ds(s * ZROWS, ZROWS)])
        plsc.subcore_barrier()

        def step(b, carry):
            pltpu.sync_copy(ones_v, acc.at[dst_v.at[b]], add=True)
            return carry

        lax.fori_loop(0, NB_DEG, step, 0)
        plsc.subcore_barrier()
        outs = (out0, out1)
        for cc in range(NCORE):
            @pl.when(c == cc)
            def _():
                pltpu.sync_copy(acc.at[pl.ds(s * ZROWS, ZROWS)],
                                outs[cc].at[pl.ds(s * ZROWS, ZROWS)])

    return pl.kernel(
        body,
        mesh=_sc_mesh(),
        out_type=[jax.ShapeDtypeStruct((NPAD,), jnp.float32)
                  for _ in range(NCORE)],
        scratch_types=[
            pltpu.VMEM((NB_DEG, EB), jnp.int32),
            pltpu.VMEM((EB,), jnp.float32),
            pltpu.VMEM_SHARED((NPAD,), jnp.float32),
            pltpu.SemaphoreType.DMA,
        ],
    )


@functools.cache
def _make_agg(nc):
    ncp = nc // NCORE

    def body(*refs):
        feat = refs[:nc]
        src_hbm, dst_hbm, zeros_hbm = refs[nc], refs[nc + 1], refs[nc + 2]
        outs = refs[nc + 3:nc + 3 + nc]
        scratch = refs[nc + 3 + nc:]
        src_sb, dst_sb, rows0, rows1, acc, sem_g, sem_s = scratch
        rows = (rows0, rows1)
        c = lax.axis_index("c")
        s = lax.axis_index("s")
        for cc in range(NCORE):
            @pl.when(c == cc)
            def _():
                for j in range(ncp):
                    ci = cc * ncp + j
                    pltpu.sync_copy(zeros_hbm, acc.at[pl.ds(s * ZROWS, ZROWS)])
                    plsc.subcore_barrier()

                    def sb_step(sb, carry, ci=ci):
                        pltpu.sync_copy(src_hbm.at[s * NSB + sb], src_sb)
                        pltpu.sync_copy(dst_hbm.at[s * NSB + sb], dst_sb)
                        h = pltpu.async_copy(feat[ci].at[src_sb.at[0]],
                                             rows[0], sem_g)
                        sc_prev = None
                        for k in range(SB):
                            h.wait()
                            sc = pltpu.async_copy(rows[k % 2],
                                                  acc.at[dst_sb.at[k]],
                                                  sem_s, priority=1,
                                                  add=True)
                            if sc_prev is not None:
                                sc_prev.wait()
                            if k + 1 < SB:
                                h = pltpu.async_copy(
                                    feat[ci].at[src_sb.at[k + 1]],
                                    rows[(k + 1) % 2], sem_g)
                            sc_prev = sc
                        sc_prev.wait()
                        return carry

                    lax.fori_loop(0, NSB, sb_step, 0)
                    plsc.subcore_barrier()
                    pltpu.sync_copy(acc.at[pl.ds(s * ZROWS, ZROWS)],
                                    outs[ci].at[pl.ds(s * ZROWS, ZROWS)])
                    plsc.subcore_barrier()

    return pl.kernel(
        body,
        mesh=_sc_mesh(),
        out_type=[jax.ShapeDtypeStruct((NPAD, CW), jnp.float32)
                  for _ in range(nc)],
        scratch_types=[
            pltpu.VMEM((SB, EB), jnp.int32),
            pltpu.VMEM((SB, EB), jnp.int32),
            pltpu.VMEM((EB, CW), jnp.float32),
            pltpu.VMEM((EB, CW), jnp.float32),
            pltpu.VMEM_SHARED((NPAD, CW), jnp.float32),
            pltpu.SemaphoreType.DMA,
            pltpu.SemaphoreType.DMA,
        ],
    )


BN = 1000


def _pre_body(degp_ref, x_ref, dis_ref, *outs):
    deg = jnp.sum(degp_ref[...], axis=1, keepdims=True) + 1.0
    dis = lax.rsqrt(deg)
    dis_ref[...] = dis
    xs = x_ref[...] * dis
    for k in range(NC1):
        outs[k][...] = xs[:, k * CW:(k + 1) * CW]


def _mm1_body(*refs):
    s1 = refs[:NC1]
    xs = refs[NC1:2 * NC1]
    dis_ref, w1t_ref, b1_ref = refs[2 * NC1:2 * NC1 + 3]
    outs = refs[2 * NC1 + 3:]
    dis = dis_ref[...]
    w = w1t_ref[...]
    acc = b1_ref[...] * jnp.ones((BN, 1), jnp.float32)
    for k in range(NC1):
        y = dis * (s1[k][...] + xs[k][...])
        acc += jnp.dot(y, w[k * CW:(k + 1) * CW, :],
                       preferred_element_type=jnp.float32)
    h = jax.nn.relu(acc)
    h1s = dis * h
    for k in range(NC2):
        outs[k][...] = h1s[:, k * CW:(k + 1) * CW]


def _mm2_body(*refs):
    s2 = refs[:NC2]
    h1s = refs[NC2:2 * NC2]
    dis_ref, w2t_ref, b2_ref, batch_ref, wct_ref, bc_ref = refs[2 * NC2:2 * NC2 + 6]
    out_ref = refs[2 * NC2 + 6]
    pooled, counts = refs[2 * NC2 + 7:]
    i = pl.program_id(0)

    @pl.when(i == 0)
    def _():
        pooled[...] = jnp.zeros_like(pooled)
        counts[...] = jnp.zeros_like(counts)

    dis = dis_ref[...]
    w = w2t_ref[...]
    acc = b2_ref[...] * jnp.ones((BN, 1), jnp.float32)
    for k in range(NC2):
        y = dis * (s2[k][...] + h1s[k][...])
        acc += jnp.dot(y, w[k * CW:(k + 1) * CW, :],
                       preferred_element_type=jnp.float32)
    h2 = jax.nn.relu(acc)
    gids = lax.broadcasted_iota(jnp.int32, (1, G), 1)
    onehot = (batch_ref[...] == gids).astype(jnp.float32)
    dn = (((0,), (0,)), ((), ()))
    pooled[...] += lax.dot_general(onehot, h2, dn,
                                   preferred_element_type=jnp.float32)
    counts[...] += lax.dot_general(onehot, jnp.ones((BN, 1), jnp.float32), dn,
                                    preferred_element_type=jnp.float32)

    @pl.when(i == pl.num_programs(0) - 1)
    def _():
        hg = pooled[...] / jnp.maximum(counts[...], 1.0)
        out_ref[...] = jnp.dot(hg, wct_ref[...],
                               preferred_element_type=jnp.float32) + bc_ref[...]


def _row_spec(width):
    return pl.BlockSpec((BN, width), lambda i: (i, 0))


def _full_spec(shape):
    return pl.BlockSpec(shape, lambda i: tuple(0 for _ in shape))


_pre_call = pl.pallas_call(
    _pre_body,
    grid=(N // BN,),
    in_specs=[_row_spec(2), _row_spec(IN)],
    out_specs=[_row_spec(1)] + [_row_spec(CW)] * NC1,
    out_shape=[jax.ShapeDtypeStruct((N, 1), jnp.float32)]
    + [jax.ShapeDtypeStruct((N, CW), jnp.float32)] * NC1,
)

_mm1_call = pl.pallas_call(
    _mm1_body,
    grid=(N // BN,),
    in_specs=[_row_spec(CW)] * (2 * NC1) + [_row_spec(1),
                                            _full_spec((IN, HID)),
                                            _full_spec((1, HID))],
    out_specs=[_row_spec(CW)] * NC2,
    out_shape=[jax.ShapeDtypeStruct((N, CW), jnp.float32)] * NC2,
)

_mm2_call = pl.pallas_call(
    _mm2_body,
    grid=(N // BN,),
    in_specs=[_row_spec(CW)] * (2 * NC2) + [
        _row_spec(1),
        _full_spec((HID, HID)),
        _full_spec((1, HID)),
        _row_spec(1),
        _full_spec((HID, OUT)),
        _full_spec((1, OUT)),
    ],
    out_specs=_full_spec((G, OUT)),
    out_shape=jax.ShapeDtypeStruct((G, OUT), jnp.float32),
    scratch_shapes=[
        pltpu.VMEM((G, HID), jnp.float32),
        pltpu.VMEM((G, 1), jnp.float32),
    ],
    compiler_params=pltpu.CompilerParams(
        dimension_semantics=("arbitrary",),
    ),
)


def kernel(x, edge_index, batch, W1, b1, W2, b2, Wc, bc):
    src = edge_index[0].reshape(NSUB * NSB, SB, EB)
    dst = edge_index[1].reshape(NSUB * NSB, SB, EB)
    dst32 = edge_index[1].reshape(NCORE * NSUB, NB_DEG, EB)
    ones_eb = jnp.ones((EB,), jnp.float32)
    zeros1 = jnp.zeros((ZROWS,), jnp.float32)
    zeros2 = jnp.zeros((ZROWS, CW), jnp.float32)

    d0, d1 = _make_deg()(dst32, ones_eb, zeros1)
    degp = jnp.stack([d0[:N], d1[:N]], axis=1)

    pre_out = _pre_call(degp, x)
    dis, xs = pre_out[0], pre_out[1:]
    s1 = _make_agg(NC1)(*xs, src, dst, zeros2)

    w1t = W1.T
    b1r = b1.reshape(1, HID)
    h1s = _mm1_call(*s1, *xs, dis, w1t, b1r)

    s2 = _make_agg(NC2)(*h1s, src, dst, zeros2)

    w2t = W2.T
    b2r = b2.reshape(1, HID)
    wct = Wc.T
    bcr = bc.reshape(1, OUT)
    batchi = batch.reshape(N, 1)
    out = _mm2_call(*s2, *h1s, dis, w2t, b2r, batchi, wct, bcr)
    return out


# speedup vs baseline: 1.0050x; 1.0050x over previous
import functools

import jax
import jax.numpy as jnp
from jax import lax
from jax.experimental import pallas as pl
from jax.experimental.pallas import tpu as pltpu
from jax.experimental.pallas import tpu_sc as plsc

N = 10000
E = 160000
IN = 256
HID = 1024
OUT = 128
G = 16

NPAD = 10240
CW = 128
EB = 100
SB = 20
NSB = 5
NB_DEG = 50
NSUB = 16
NCORE = 2
ZROWS = NPAD // NSUB
NC1 = IN // CW
NC2 = HID // CW


@functools.cache
def _sc_mesh():
    return plsc.VectorSubcoreMesh(core_axis_name="c", subcore_axis_name="s",
                                  num_cores=NCORE, num_subcores=NSUB)


@functools.cache
def _make_deg():
    def body(dst_hbm, ones_hbm, zeros_hbm, out0, out1, dst_v, ones_v, acc, sem):
        c = lax.axis_index("c")
        s = lax.axis_index("s")
        pltpu.sync_copy(dst_hbm.at[c * NSUB + s], dst_v)
        pltpu.sync_copy(ones_hbm, ones_v)
        pltpu.sync_copy(zeros_hbm, acc.at[pl.ds(s * ZROWS, ZROWS)])
        plsc.subcore_barrier()

        def step(b, carry):
            pltpu.sync_copy(ones_v, acc.at[dst_v.at[b]], add=True)
            return carry

        lax.fori_loop(0, NB_DEG, step, 0)
        plsc.subcore_barrier()
        outs = (out0, out1)
        for cc in range(NCORE):
            @pl.when(c == cc)
            def _():
                pltpu.sync_copy(acc.at[pl.ds(s * ZROWS, ZROWS)],
                                outs[cc].at[pl.ds(s * ZROWS, ZROWS)])

    return pl.kernel(
        body,
        mesh=_sc_mesh(),
        out_type=[jax.ShapeDtypeStruct((NPAD,), jnp.float32)
                  for _ in range(NCORE)],
        scratch_types=[
            pltpu.VMEM((NB_DEG, EB), jnp.int32),
            pltpu.VMEM((EB,), jnp.float32),
            pltpu.VMEM_SHARED((NPAD,), jnp.float32),
            pltpu.SemaphoreType.DMA,
        ],
    )


@functools.cache
def _make_agg(nc, dtype):
    ncp = nc // NCORE

    def body(*refs):
        feat = refs[:nc]
        src_hbm, dst_hbm, zeros_hbm = refs[nc], refs[nc + 1], refs[nc + 2]
        outs = refs[nc + 3:nc + 3 + nc]
        scratch = refs[nc + 3 + nc:]
        src_sb, dst_sb, rows0, rows1, acc, sem_g, sem_s = scratch
        rows = (rows0, rows1)
        c = lax.axis_index("c")
        s = lax.axis_index("s")
        for cc in range(NCORE):
            @pl.when(c == cc)
            def _():
                for j in range(ncp):
                    ci = cc * ncp + j
                    pltpu.sync_copy(zeros_hbm, acc.at[pl.ds(s * ZROWS, ZROWS)])
                    plsc.subcore_barrier()

                    def sb_step(sb, carry, ci=ci):
                        pltpu.sync_copy(src_hbm.at[s * NSB + sb], src_sb)
                        pltpu.sync_copy(dst_hbm.at[s * NSB + sb], dst_sb)
                        h = pltpu.async_copy(feat[ci].at[src_sb.at[0]],
                                             rows[0], sem_g)
                        sc_prev = None
                        for k in range(SB):
                            h.wait()
                            sc = pltpu.async_copy(rows[k % 2],
                                                  acc.at[dst_sb.at[k]],
                                                  sem_s, priority=1,
                                                  add=True)
                            if sc_prev is not None:
                                sc_prev.wait()
                            if k + 1 < SB:
                                h = pltpu.async_copy(
                                    feat[ci].at[src_sb.at[k + 1]],
                                    rows[(k + 1) % 2], sem_g)
                            sc_prev = sc
                        sc_prev.wait()
                        return carry

                    lax.fori_loop(0, NSB, sb_step, 0)
                    plsc.subcore_barrier()
                    pltpu.sync_copy(acc.at[pl.ds(s * ZROWS, ZROWS)],
                                    outs[ci].at[pl.ds(s * ZROWS, ZROWS)])
                    plsc.subcore_barrier()

    return pl.kernel(
        body,
        mesh=_sc_mesh(),
        out_type=[jax.ShapeDtypeStruct((NPAD, CW), dtype)
                  for _ in range(nc)],
        scratch_types=[
            pltpu.VMEM((SB, EB), jnp.int32),
            pltpu.VMEM((SB, EB), jnp.int32),
            pltpu.VMEM((EB, CW), dtype),
            pltpu.VMEM((EB, CW), dtype),
            pltpu.VMEM_SHARED((NPAD, CW), dtype),
            pltpu.SemaphoreType.DMA,
            pltpu.SemaphoreType.DMA,
        ],
    )


BN = 1000


def _pre_body(degp_ref, x_ref, dis_ref, *outs):
    deg = jnp.sum(degp_ref[...], axis=1, keepdims=True) + 1.0
    dis = lax.rsqrt(deg)
    dis_ref[...] = dis
    xs = x_ref[...] * dis
    for k in range(NC1):
        outs[k][...] = xs[:, k * CW:(k + 1) * CW]


def _mm1_body(*refs):
    s1 = refs[:NC1]
    xs = refs[NC1:2 * NC1]
    dis_ref, w1t_ref, b1_ref = refs[2 * NC1:2 * NC1 + 3]
    outs = refs[2 * NC1 + 3:]
    dis = dis_ref[...]
    w = w1t_ref[...]
    acc = b1_ref[...] * jnp.ones((BN, 1), jnp.float32)
    for k in range(NC1):
        y = dis * (s1[k][...] + xs[k][...])
        acc += jnp.dot(y.astype(jnp.bfloat16), w[k * CW:(k + 1) * CW, :],
                       preferred_element_type=jnp.float32)
    h = jax.nn.relu(acc)
    h1s = dis * h
    for k in range(NC2):
        outs[k][...] = h1s[:, k * CW:(k + 1) * CW]


def _mm2_body(*refs):
    s2 = refs[:NC2]
    h1s = refs[NC2:2 * NC2]
    dis_ref, w2t_ref, b2_ref, batch_ref, wct_ref, bc_ref = refs[2 * NC2:2 * NC2 + 6]
    out_ref = refs[2 * NC2 + 6]
    pooled, counts = refs[2 * NC2 + 7:]
    i = pl.program_id(0)

    @pl.when(i == 0)
    def _():
        pooled[...] = jnp.zeros_like(pooled)
        counts[...] = jnp.zeros_like(counts)

    dis = dis_ref[...]
    w = w2t_ref[...]
    acc = b2_ref[...] * jnp.ones((BN, 1), jnp.float32)
    for k in range(NC2):
        y = dis * (s2[k][...] + h1s[k][...])
        acc += jnp.dot(y.astype(jnp.bfloat16), w[k * CW:(k + 1) * CW, :],
                       preferred_element_type=jnp.float32)
    h2 = jax.nn.relu(acc)
    gids = lax.broadcasted_iota(jnp.int32, (1, G), 1)
    onehot = (batch_ref[...] == gids).astype(jnp.float32)
    dn = (((0,), (0,)), ((), ()))
    pooled[...] += lax.dot_general(onehot, h2, dn,
                                   preferred_element_type=jnp.float32)
    counts[...] += lax.dot_general(onehot, jnp.ones((BN, 1), jnp.float32), dn,
                                    preferred_element_type=jnp.float32)

    @pl.when(i == pl.num_programs(0) - 1)
    def _():
        hg = pooled[...] / jnp.maximum(counts[...], 1.0)
        out_ref[...] = jnp.dot(hg, wct_ref[...],
                               preferred_element_type=jnp.float32) + bc_ref[...]


def _row_spec(width):
    return pl.BlockSpec((BN, width), lambda i: (i, 0))


def _full_spec(shape):
    return pl.BlockSpec(shape, lambda i: tuple(0 for _ in shape))


_pre_call = pl.pallas_call(
    _pre_body,
    grid=(N // BN,),
    in_specs=[_row_spec(2), _row_spec(IN)],
    out_specs=[_row_spec(1)] + [_row_spec(CW)] * NC1,
    out_shape=[jax.ShapeDtypeStruct((N, 1), jnp.float32)]
    + [jax.ShapeDtypeStruct((N, CW), jnp.float32)] * NC1,
)

_mm1_call = pl.pallas_call(
    _mm1_body,
    grid=(N // BN,),
    in_specs=[_row_spec(CW)] * (2 * NC1) + [_row_spec(1),
                                            _full_spec((IN, HID)),
                                            _full_spec((1, HID))],
    out_specs=[_row_spec(CW)] * NC2,
    out_shape=[jax.ShapeDtypeStruct((N, CW), jnp.float32)] * NC2,
)

_mm2_call = pl.pallas_call(
    _mm2_body,
    grid=(N // BN,),
    in_specs=[_row_spec(CW)] * (2 * NC2) + [
        _row_spec(1),
        _full_spec((HID, HID)),
        _full_spec((1, HID)),
        _row_spec(1),
        _full_spec((HID, OUT)),
        _full_spec((1, OUT)),
    ],
    out_specs=_full_spec((G, OUT)),
    out_shape=jax.ShapeDtypeStruct((G, OUT), jnp.float32),
    scratch_shapes=[
        pltpu.VMEM((G, HID), jnp.float32),
        pltpu.VMEM((G, 1), jnp.float32),
    ],
    compiler_params=pltpu.CompilerParams(
        dimension_semantics=("arbitrary",),
    ),
)


def kernel(x, edge_index, batch, W1, b1, W2, b2, Wc, bc):
    src = edge_index[0].reshape(NSUB * NSB, SB, EB)
    dst = edge_index[1].reshape(NSUB * NSB, SB, EB)
    dst32 = edge_index[1].reshape(NCORE * NSUB, NB_DEG, EB)
    ones_eb = jnp.ones((EB,), jnp.float32)
    zeros1 = jnp.zeros((ZROWS,), jnp.float32)
    zeros2 = jnp.zeros((ZROWS, CW), jnp.float32)

    d0, d1 = _make_deg()(dst32, ones_eb, zeros1)
    degp = jnp.stack([d0[:N], d1[:N]], axis=1)

    pre_out = _pre_call(degp, x)
    dis, xs = pre_out[0], pre_out[1:]
    s1 = _make_agg(NC1, jnp.float32)(*xs, src, dst, zeros2)

    w1t = W1.T.astype(jnp.bfloat16)
    b1r = b1.reshape(1, HID)
    h1s = _mm1_call(*s1, *xs, dis, w1t, b1r)

    s2 = _make_agg(NC2, jnp.float32)(*h1s, src, dst, zeros2)

    w2t = W2.T.astype(jnp.bfloat16)
    b2r = b2.reshape(1, HID)
    wct = Wc.T
    bcr = bc.reshape(1, OUT)
    batchi = batch.reshape(N, 1)
    out = _mm2_call(*s2, *h1s, dis, w2t, b2r, batchi, wct, bcr)
    return out


# speedup vs baseline: 1.3399x; 1.3332x over previous
import functools

import jax
import jax.numpy as jnp
from jax import lax
from jax.experimental import pallas as pl
from jax.experimental.pallas import tpu as pltpu
from jax.experimental.pallas import tpu_sc as plsc

N = 10000
E = 160000
IN = 256
HID = 1024
OUT = 128
G = 16

NPAD = 10240
CW = 128
EB = 100
SB = 20
NSB = 5
NB_DEG = 50
NSUB = 16
NCORE = 2
ZROWS = NPAD // NSUB
NC1 = IN // CW
NC2 = HID // CW


@functools.cache
def _sc_mesh():
    return plsc.VectorSubcoreMesh(core_axis_name="c", subcore_axis_name="s",
                                  num_cores=NCORE, num_subcores=NSUB)


@functools.cache
def _make_deg():
    def body(dst_hbm, ones_hbm, zeros_hbm, out0, out1, dst_v, ones_v, acc, sem):
        c = lax.axis_index("c")
        s = lax.axis_index("s")
        pltpu.sync_copy(dst_hbm.at[c * NSUB + s], dst_v)
        pltpu.sync_copy(ones_hbm, ones_v)
        pltpu.sync_copy(zeros_hbm, acc.at[pl.ds(s * ZROWS, ZROWS)])
        plsc.subcore_barrier()

        def step(b, carry):
            pltpu.sync_copy(ones_v, acc.at[dst_v.at[b]], add=True)
            return carry

        lax.fori_loop(0, NB_DEG, step, 0)
        plsc.subcore_barrier()
        outs = (out0, out1)
        for cc in range(NCORE):
            @pl.when(c == cc)
            def _():
                pltpu.sync_copy(acc.at[pl.ds(s * ZROWS, ZROWS)],
                                outs[cc].at[pl.ds(s * ZROWS, ZROWS)])

    return pl.kernel(
        body,
        mesh=_sc_mesh(),
        out_type=[jax.ShapeDtypeStruct((NPAD,), jnp.float32)
                  for _ in range(NCORE)],
        scratch_types=[
            pltpu.VMEM((NB_DEG, EB), jnp.int32),
            pltpu.VMEM((EB,), jnp.float32),
            pltpu.VMEM_SHARED((NPAD,), jnp.float32),
            pltpu.SemaphoreType.DMA,
        ],
    )


@functools.cache
def _make_agg(nc, dtype):
    ncp = nc // NCORE

    def body(*refs):
        feat = refs[:nc]
        src_hbm, dst_hbm, zeros_hbm = refs[nc], refs[nc + 1], refs[nc + 2]
        outs = refs[nc + 3:nc + 3 + nc]
        scratch = refs[nc + 3 + nc:]
        src_sb, dst_sb, rows0, rows1, rows2, acc, sem_g, sem_s = scratch
        rows = (rows0, rows1, rows2)
        c = lax.axis_index("c")
        s = lax.axis_index("s")
        for cc in range(NCORE):
            @pl.when(c == cc)
            def _():
                for j in range(ncp):
                    ci = cc * ncp + j
                    pltpu.sync_copy(zeros_hbm, acc.at[pl.ds(s * ZROWS, ZROWS)])
                    plsc.subcore_barrier()

                    def sb_step(sb, carry, ci=ci):
                        pltpu.sync_copy(src_hbm.at[s * NSB + sb], src_sb)
                        pltpu.sync_copy(dst_hbm.at[s * NSB + sb], dst_sb)
                        hs = [pltpu.async_copy(feat[ci].at[src_sb.at[k]],
                                               rows[k], sem_g)
                              for k in range(2)]
                        for k in range(SB):
                            hs[k].wait()
                            if k + 2 < SB:
                                hs.append(pltpu.async_copy(
                                    feat[ci].at[src_sb.at[k + 2]],
                                    rows[(k + 2) % 3], sem_g))
                            pltpu.sync_copy(rows[k % 3],
                                            acc.at[dst_sb.at[k]], add=True)
                        return carry

                    lax.fori_loop(0, NSB, sb_step, 0)
                    plsc.subcore_barrier()
                    pltpu.sync_copy(acc.at[pl.ds(s * ZROWS, ZROWS)],
                                    outs[ci].at[pl.ds(s * ZROWS, ZROWS)])
                    plsc.subcore_barrier()

    return pl.kernel(
        body,
        mesh=_sc_mesh(),
        out_type=[jax.ShapeDtypeStruct((NPAD, CW), dtype)
                  for _ in range(nc)],
        scratch_types=[
            pltpu.VMEM((SB, EB), jnp.int32),
            pltpu.VMEM((SB, EB), jnp.int32),
            pltpu.VMEM((EB, CW), dtype),
            pltpu.VMEM((EB, CW), dtype),
            pltpu.VMEM((EB, CW), dtype),
            pltpu.VMEM_SHARED((NPAD, CW), dtype),
            pltpu.SemaphoreType.DMA,
            pltpu.SemaphoreType.DMA,
        ],
    )


BN = 1000


def _pre_body(degp_ref, x_ref, dis_ref, *outs):
    deg = jnp.sum(degp_ref[...], axis=1, keepdims=True) + 1.0
    dis = lax.rsqrt(deg)
    dis_ref[...] = dis
    xs = x_ref[...] * dis
    for k in range(NC1):
        outs[k][...] = xs[:, k * CW:(k + 1) * CW]


def _mm1_body(*refs):
    s1 = refs[:NC1]
    xs = refs[NC1:2 * NC1]
    dis_ref, w1t_ref, b1_ref = refs[2 * NC1:2 * NC1 + 3]
    outs = refs[2 * NC1 + 3:]
    dis = dis_ref[...]
    w = w1t_ref[...]
    acc = b1_ref[...] * jnp.ones((BN, 1), jnp.float32)
    for k in range(NC1):
        y = dis * (s1[k][...] + xs[k][...])
        acc += jnp.dot(y.astype(jnp.bfloat16), w[k * CW:(k + 1) * CW, :],
                       preferred_element_type=jnp.float32)
    h = jax.nn.relu(acc)
    h1s = dis * h
    for k in range(NC2):
        outs[k][...] = h1s[:, k * CW:(k + 1) * CW]


def _mm2_body(*refs):
    s2 = refs[:NC2]
    h1s = refs[NC2:2 * NC2]
    dis_ref, w2t_ref, b2_ref, batch_ref, wct_ref, bc_ref = refs[2 * NC2:2 * NC2 + 6]
    out_ref = refs[2 * NC2 + 6]
    pooled, counts = refs[2 * NC2 + 7:]
    i = pl.program_id(0)

    @pl.when(i == 0)
    def _():
        pooled[...] = jnp.zeros_like(pooled)
        counts[...] = jnp.zeros_like(counts)

    dis = dis_ref[...]
    w = w2t_ref[...]
    acc = b2_ref[...] * jnp.ones((BN, 1), jnp.float32)
    for k in range(NC2):
        y = dis * (s2[k][...] + h1s[k][...])
        acc += jnp.dot(y.astype(jnp.bfloat16), w[k * CW:(k + 1) * CW, :],
                       preferred_element_type=jnp.float32)
    h2 = jax.nn.relu(acc)
    gids = lax.broadcasted_iota(jnp.int32, (1, G), 1)
    onehot = (batch_ref[...] == gids).astype(jnp.float32)
    dn = (((0,), (0,)), ((), ()))
    pooled[...] += lax.dot_general(onehot, h2, dn,
                                   preferred_element_type=jnp.float32)
    counts[...] += lax.dot_general(onehot, jnp.ones((BN, 1), jnp.float32), dn,
                                    preferred_element_type=jnp.float32)

    @pl.when(i == pl.num_programs(0) - 1)
    def _():
        hg = pooled[...] / jnp.maximum(counts[...], 1.0)
        out_ref[...] = jnp.dot(hg, wct_ref[...],
                               preferred_element_type=jnp.float32) + bc_ref[...]


def _row_spec(width):
    return pl.BlockSpec((BN, width), lambda i: (i, 0))


def _full_spec(shape):
    return pl.BlockSpec(shape, lambda i: tuple(0 for _ in shape))


_pre_call = pl.pallas_call(
    _pre_body,
    grid=(N // BN,),
    in_specs=[_row_spec(2), _row_spec(IN)],
    out_specs=[_row_spec(1)] + [_row_spec(CW)] * NC1,
    out_shape=[jax.ShapeDtypeStruct((N, 1), jnp.float32)]
    + [jax.ShapeDtypeStruct((N, CW), jnp.float32)] * NC1,
)

_mm1_call = pl.pallas_call(
    _mm1_body,
    grid=(N // BN,),
    in_specs=[_row_spec(CW)] * (2 * NC1) + [_row_spec(1),
                                            _full_spec((IN, HID)),
                                            _full_spec((1, HID))],
    out_specs=[_row_spec(CW)] * NC2,
    out_shape=[jax.ShapeDtypeStruct((N, CW), jnp.float32)] * NC2,
)

_mm2_call = pl.pallas_call(
    _mm2_body,
    grid=(N // BN,),
    in_specs=[_row_spec(CW)] * (2 * NC2) + [
        _row_spec(1),
        _full_spec((HID, HID)),
        _full_spec((1, HID)),
        _row_spec(1),
        _full_spec((HID, OUT)),
        _full_spec((1, OUT)),
    ],
    out_specs=_full_spec((G, OUT)),
    out_shape=jax.ShapeDtypeStruct((G, OUT), jnp.float32),
    scratch_shapes=[
        pltpu.VMEM((G, HID), jnp.float32),
        pltpu.VMEM((G, 1), jnp.float32),
    ],
    compiler_params=pltpu.CompilerParams(
        dimension_semantics=("arbitrary",),
    ),
)


def kernel(x, edge_index, batch, W1, b1, W2, b2, Wc, bc):
    src = edge_index[0].reshape(NSUB * NSB, SB, EB)
    dst = edge_index[1].reshape(NSUB * NSB, SB, EB)
    dst32 = edge_index[1].reshape(NCORE * NSUB, NB_DEG, EB)
    ones_eb = jnp.ones((EB,), jnp.float32)
    zeros1 = jnp.zeros((ZROWS,), jnp.float32)
    zeros2 = jnp.zeros((ZROWS, CW), jnp.float32)

    d0, d1 = _make_deg()(dst32, ones_eb, zeros1)
    degp = jnp.stack([d0[:N], d1[:N]], axis=1)

    pre_out = _pre_call(degp, x)
    dis, xs = pre_out[0], pre_out[1:]
    s1 = _make_agg(NC1, jnp.float32)(*xs, src, dst, zeros2)

    w1t = W1.T.astype(jnp.bfloat16)
    b1r = b1.reshape(1, HID)
    h1s = _mm1_call(*s1, *xs, dis, w1t, b1r)

    s2 = _make_agg(NC2, jnp.float32)(*h1s, src, dst, zeros2)

    w2t = W2.T.astype(jnp.bfloat16)
    b2r = b2.reshape(1, HID)
    wct = Wc.T
    bcr = bc.reshape(1, OUT)
    batchi = batch.reshape(N, 1)
    out = _mm2_call(*s2, *h1s, dis, w2t, b2r, batchi, wct, bcr)
    return out
